# Initial kernel scaffold; baseline (speedup 1.0000x reference)
import jax
import jax.numpy as jnp
from jax.experimental import pallas as pl


def kernel(x, sem_w, sem_b, q_w, k_w, v_w, fuse_w, fuse_b, alpha, ln_w, ln_b):
    raise NotImplementedError("write your pallas kernel here")


# speedup vs baseline: 2.9639x; 2.9639x over previous
import jax
import jax.numpy as jnp
from jax import lax
from jax.experimental import pallas as pl
from jax.experimental.pallas import tpu as pltpu

B, C, H, W = 2, 192, 224, 224
K = 16
TOPK = 8
DS = 4
EPS = 1e-6
D_INNER = C // 4
Hd, Wd = H // DS, W // DS
Nd = Hd * Wd
M = 256
CT = 32
NT = 448
HT = 32
NEG = -1e30
F32 = jnp.float32


def _fiota(shape, dim):
    return lax.broadcasted_iota(jnp.int32, shape, dim).astype(F32)


def _dn_kernel(x_ref, o_ref):
    xb = x_ref[0]
    ip = _fiota((Wd, W), 0)
    jp = _fiota((Wd, W), 1)
    P = (jnp.floor(jp * (1.0 / DS)) == ip).astype(F32) * (1.0 / DS)
    a = xb.reshape(CT * H, W)
    t0 = lax.dot_general(a, P, (((1,), (1,)), ((), ())),
                         preferred_element_type=F32)
    t0 = t0.reshape(CT, H, Wd).transpose(1, 0, 2).reshape(H, CT * Wd)
    t1 = jnp.dot(P, t0, preferred_element_type=F32)
    t1 = t1.reshape(Hd, CT, Wd).transpose(1, 0, 2)
    o_ref[0] = t1.reshape(CT, Nd)


def _softmax(x, axis):
    m = jnp.max(x, axis=axis, keepdims=True)
    e = jnp.exp(x - m)
    return e / jnp.sum(e, axis=axis, keepdims=True)


def _global_kernel(xf_ref, semw_ref, semb_ref, qw_ref, kw_ref, vw_ref,
                   qT_ref, vT_ref, rk_ref, rv_ref, R_ref, kp_ref, ti_ref,
                   work_ref, tis_ref):
    xf = xf_ref[0]
    dn = (((0,), (1,)), ((), ()))
    semT = lax.dot_general(xf, semw_ref[0], dn, preferred_element_type=F32)
    semT = semT + semb_ref[0]
    semT = _softmax(semT, axis=1)
    qT = lax.dot_general(xf, qw_ref[0], dn, preferred_element_type=F32)
    kT = lax.dot_general(xf, kw_ref[0], dn, preferred_element_type=F32)
    vT = lax.dot_general(xf, vw_ref[0], dn, preferred_element_type=F32)
    qT_ref[0] = qT
    vT_ref[0] = vT
    dn0 = (((0,), (0,)), ((), ()))
    rk_ref[0] = lax.dot_general(semT, kT, dn0, preferred_element_type=F32)
    rv_ref[0] = lax.dot_general(semT, vT, dn0, preferred_element_type=F32)

    iota_k = _fiota((Nd, K), 1)
    mx = jnp.max(semT, axis=1, keepdims=True)
    ridx = jnp.min(jnp.where(semT == mx, iota_k, float(K)), axis=1,
                   keepdims=True)
    R_ref[0] = (iota_k == ridx).astype(F32)

    work_ref[...] = semT
    iota_n = _fiota((Nd, K), 0)

    def body(m, _):
        w = work_ref[...]
        val = jnp.max(w, axis=0, keepdims=True)
        idx = jnp.min(jnp.where(w == val, iota_n, float(Nd)), axis=0,
                      keepdims=True)
        tis_ref[pl.ds(m, 1), :] = idx
        work_ref[...] = jnp.where(iota_n == idx, NEG, w)
        return 0

    lax.fori_loop(0, M, body, 0)
    tis = tis_ref[...]
    ti_ref[0] = tis.T

    iota_row = _fiota((M, Nd), 1)
    for r in range(K):
        g = (tis[:, r:r + 1] == iota_row).astype(F32)
        kp_ref[0, r * M:(r + 1) * M, :] = jnp.dot(
            g, kT, preferred_element_type=F32)


def _attn_kernel(q_ref, R_ref, kp_ref, ti_ref, rk_ref, rv_ref, v_ref,
                 alpha_ref, fw_ref, fb_ref, o_ref):
    q = q_ref[0]
    Rm = R_ref[0]
    kp = kp_ref[0]
    ti = ti_ref[0]
    vT = v_ref[0]
    scale = float(D_INNER) ** 0.5 + EPS
    dn11 = (((1,), (1,)), ((), ()))

    sreg = lax.dot_general(q, rk_ref[0], dn11,
                           preferred_element_type=F32) / scale
    attn = _softmax(sreg, axis=1)
    cont = jnp.dot(attn, rv_ref[0], preferred_element_type=F32)

    sall = lax.dot_general(q, kp, dn11,
                           preferred_element_type=F32) / scale
    sel = jnp.zeros((NT, M), dtype=F32)
    tsel = jnp.zeros((NT, M), dtype=F32)
    for r in range(K):
        sel = sel + Rm[:, r:r + 1] * sall[:, r * M:(r + 1) * M]
        tsel = tsel + Rm[:, r:r + 1] * ti[r:r + 1, :]

    iota_m = _fiota((NT, M), 1)
    work = sel
    topv = []
    topi = []
    for _ in range(TOPK):
        val = jnp.max(work, axis=1, keepdims=True)
        idx = jnp.min(jnp.where(work == val, iota_m, float(M)), axis=1,
                      keepdims=True)
        topv.append(val)
        topi.append(idx)
        work = jnp.where(iota_m == idx, NEG, work)
    att = _softmax(jnp.concatenate(topv, axis=1), axis=1)

    iota_n = _fiota((NT, Nd), 1)
    acc = jnp.zeros((NT, Nd), dtype=F32)
    for j in range(TOPK):
        nsrc = jnp.sum(tsel * (iota_m == topi[j]), axis=1,
                       keepdims=True)
        acc = acc + att[:, j:j + 1] * (iota_n == nsrc).astype(F32)
    sparse = jnp.dot(acc, vT, preferred_element_type=F32)

    a = jax.nn.sigmoid(alpha_ref[0, 0])
    comb = a * cont + (1.0 - a) * sparse
    fused = lax.dot_general(comb, fw_ref[0], dn11,
                            preferred_element_type=F32) + fb_ref[0]
    o_ref[0] = fused


def _interp_mat(rows, cols, row0):
    i = _fiota((rows, cols), 0) + row0
    j = _fiota((rows, cols), 1)
    src = i * (1.0 / DS) - 0.375
    i0 = jnp.floor(src)
    w1 = src - i0
    j0 = jnp.clip(i0, 0.0, float(cols - 1))
    j1 = jnp.clip(i0 + 1.0, 0.0, float(cols - 1))
    return (j == j0).astype(F32) * (1.0 - w1) + (j == j1).astype(F32) * w1


def _up_kernel(f_ref, x_ref, lnw_ref, lnb_ref, o_ref):
    h0 = (pl.program_id(1) * HT).astype(F32)
    Uw = _interp_mat(W, Wd, 0.0)
    Uh = _interp_mat(HT, Hd, h0)

    f = f_ref[0].T.reshape(C, Hd, Wd)
    f2 = f.transpose(1, 0, 2).reshape(Hd, C * Wd)
    t1 = jnp.dot(Uh, f2, preferred_element_type=F32)
    t1 = t1.reshape(HT, C, Wd).transpose(1, 0, 2).reshape(C * HT, Wd)
    t2 = lax.dot_general(t1, Uw, (((1,), (1,)), ((), ())),
                         preferred_element_type=F32)
    y = t2.reshape(C, HT, W)

    mu = jnp.mean(y, axis=0, keepdims=True)
    d = y - mu
    var = jnp.mean(d * d, axis=0, keepdims=True)
    yn = d * lax.rsqrt(var + 1e-5)
    yn = yn * lnw_ref[...].reshape(C, 1, 1) + lnb_ref[...].reshape(C, 1, 1)
    o_ref[0] = x_ref[0] + yn


def kernel(x, sem_w, sem_b, q_w, k_w, v_w, fuse_w, fuse_b, alpha, ln_w, ln_b):
    f32 = jnp.float32
    x = x.astype(f32)
    sem_b2 = sem_b.reshape(1, K).astype(f32)
    fuse_b2 = fuse_b.reshape(1, C).astype(f32)
    alpha2 = alpha.reshape(1, 1).astype(f32)
    ln_w2 = ln_w.reshape(C, 1).astype(f32)
    ln_b2 = ln_b.reshape(C, 1).astype(f32)

    xf = pl.pallas_call(
        _dn_kernel,
        grid=(B, C // CT),
        in_specs=[pl.BlockSpec((1, CT, H, W), lambda b, c: (b, c, 0, 0))],
        out_specs=pl.BlockSpec((1, CT, Nd), lambda b, c: (b, c, 0)),
        out_shape=jax.ShapeDtypeStruct((B, C, Nd), f32),
    )(x)

    full = lambda b: (b, 0, 0)
    wspec = lambda s: pl.BlockSpec((1,) + s, lambda b: (0,) + (0,) * len(s))
    qT, vT, rk, rv, R, kp, ti = pl.pallas_call(
        _global_kernel,
        grid=(B,),
        in_specs=[
            pl.BlockSpec((1, C, Nd), full),
            wspec((K, C)), wspec((1, K)), wspec((D_INNER, C)),
            wspec((D_INNER, C)), wspec((C, C)),
        ],
        out_specs=[
            pl.BlockSpec((1, Nd, D_INNER), full),
            pl.BlockSpec((1, Nd, C), full),
            pl.BlockSpec((1, K, D_INNER), full),
            pl.BlockSpec((1, K, C), full),
            pl.BlockSpec((1, Nd, K), full),
            pl.BlockSpec((1, K * M, D_INNER), full),
            pl.BlockSpec((1, K, M), full),
        ],
        out_shape=[
            jax.ShapeDtypeStruct((B, Nd, D_INNER), f32),
            jax.ShapeDtypeStruct((B, Nd, C), f32),
            jax.ShapeDtypeStruct((B, K, D_INNER), f32),
            jax.ShapeDtypeStruct((B, K, C), f32),
            jax.ShapeDtypeStruct((B, Nd, K), f32),
            jax.ShapeDtypeStruct((B, K * M, D_INNER), f32),
            jax.ShapeDtypeStruct((B, K, M), f32),
        ],
        scratch_shapes=[pltpu.VMEM((Nd, K), f32), pltpu.VMEM((M, K), f32)],
    )(xf, sem_w.reshape(1, K, C), sem_b2.reshape(1, 1, K),
      q_w.reshape(1, D_INNER, C), k_w.reshape(1, D_INNER, C),
      v_w.reshape(1, C, C))

    tile = lambda b, t: (b, t, 0)
    w2 = lambda s: pl.BlockSpec((1,) + s, lambda b, t: (0,) + (0,) * len(s))
    fullc = lambda b, t: (b, 0, 0)
    fused = pl.pallas_call(
        _attn_kernel,
        grid=(B, Nd // NT),
        in_specs=[
            pl.BlockSpec((1, NT, D_INNER), tile),
            pl.BlockSpec((1, NT, K), tile),
            pl.BlockSpec((1, K * M, D_INNER), fullc),
            pl.BlockSpec((1, K, M), fullc),
            pl.BlockSpec((1, K, D_INNER), fullc),
            pl.BlockSpec((1, K, C), fullc),
            pl.BlockSpec((1, Nd, C), fullc),
            w2((1, 1)), w2((C, C)), w2((1, C)),
        ],
        out_specs=pl.BlockSpec((1, NT, C), lambda b, t: (b, t, 0)),
        out_shape=jax.ShapeDtypeStruct((B, Nd, C), f32),
    )(qT, R, kp, ti, rk, rv, vT, alpha2.reshape(1, 1, 1),
      fuse_w.reshape(1, C, C), fuse_b2.reshape(1, 1, C))

    out = pl.pallas_call(
        _up_kernel,
        grid=(B, H // HT),
        in_specs=[
            pl.BlockSpec((1, Nd, C), lambda b, h: (b, 0, 0)),
            pl.BlockSpec((1, C, HT, W), lambda b, h: (b, 0, h, 0)),
            pl.BlockSpec((C, 1), lambda b, h: (0, 0)),
            pl.BlockSpec((C, 1), lambda b, h: (0, 0)),
        ],
        out_specs=pl.BlockSpec((1, C, HT, W), lambda b, h: (b, 0, h, 0)),
        out_shape=jax.ShapeDtypeStruct((B, C, H, W), f32),
    )(fused, x, ln_w2, ln_b2)
    return out


# speedup vs baseline: 5.1428x; 1.7351x over previous
import jax
import jax.numpy as jnp
from jax import lax
from jax.experimental import pallas as pl
from jax.experimental.pallas import tpu as pltpu

B, C, H, W = 2, 192, 224, 224
K = 16
TOPK = 8
DS = 4
EPS = 1e-6
D_INNER = C // 4
Hd, Wd = H // DS, W // DS
Nd = Hd * Wd
M = 256
CT = 32
NT = 448
HT = 32
NEG = -1e30
F32 = jnp.float32


def _fiota(shape, dim):
    return lax.broadcasted_iota(jnp.int32, shape, dim).astype(F32)


def _dn_kernel(x_ref, o_ref):
    xb = x_ref[0]
    ip = _fiota((Wd, W), 0)
    jp = _fiota((Wd, W), 1)
    P = (jnp.floor(jp * (1.0 / DS)) == ip).astype(F32) * (1.0 / DS)
    a = xb.reshape(CT * H, W)
    t0 = lax.dot_general(a, P, (((1,), (1,)), ((), ())),
                         preferred_element_type=F32)
    t0 = t0.reshape(CT, H, Wd).transpose(1, 0, 2).reshape(H, CT * Wd)
    t1 = jnp.dot(P, t0, preferred_element_type=F32)
    t1 = t1.reshape(Hd, CT, Wd).transpose(1, 0, 2)
    o_ref[0] = t1.reshape(CT, Nd)


def _softmax(x, axis):
    m = jnp.max(x, axis=axis, keepdims=True)
    e = jnp.exp(x - m)
    return e / jnp.sum(e, axis=axis, keepdims=True)


def _global_kernel(xf_ref, semw_ref, semb_ref, qw_ref, kw_ref, vw_ref,
                   qT_ref, vT_ref, rk_ref, rv_ref, R_ref, kp_ref, ti_ref,
                   work_ref, tis_ref):
    xf = xf_ref[0]
    dn = (((0,), (1,)), ((), ()))
    sem = jnp.dot(semw_ref[0], xf, preferred_element_type=F32)
    sem = _softmax(sem + semb_ref[0], axis=0)
    qT = lax.dot_general(xf, qw_ref[0], dn, preferred_element_type=F32)
    kT = lax.dot_general(xf, kw_ref[0], dn, preferred_element_type=F32)
    vT = lax.dot_general(xf, vw_ref[0], dn, preferred_element_type=F32)
    qT_ref[0] = qT
    vT_ref[0] = vT
    dnr = (((1,), (0,)), ((), ()))
    rk_ref[0] = lax.dot_general(sem, kT, dnr, preferred_element_type=F32)
    rv_ref[0] = lax.dot_general(sem, vT, dnr, preferred_element_type=F32)

    iota_kc = _fiota((K, Nd), 0)
    mx = jnp.max(sem, axis=0, keepdims=True)
    ridx = jnp.min(jnp.where(sem == mx, iota_kc, float(K)), axis=0,
                   keepdims=True)
    R_ref[0] = (_fiota((Nd, K), 1) == ridx.T).astype(F32)

    work_ref[...] = sem
    iota_nc = _fiota((K, Nd), 1)
    lane_m = lax.broadcasted_iota(jnp.int32, (K, M), 1)

    def body(m, _):
        w = work_ref[...]
        val = jnp.max(w, axis=1, keepdims=True)
        idx = jnp.min(jnp.where(w == val, iota_nc, float(Nd)), axis=1,
                      keepdims=True)
        tis_ref[...] = jnp.where(lane_m == m, idx, tis_ref[...])
        work_ref[...] = jnp.where(iota_nc == idx, NEG, w)
        return 0

    lax.fori_loop(0, M, body, 0)
    tis = tis_ref[...]
    ti_ref[0] = tis

    iota_row = _fiota((M, Nd), 1)
    for r in range(K):
        g = (tis[r:r + 1, :].T == iota_row).astype(F32)
        kp_ref[0, r * M:(r + 1) * M, :] = jnp.dot(
            g, kT, preferred_element_type=F32)


def _attn_kernel(q_ref, R_ref, kp_ref, ti_ref, rk_ref, rv_ref, v_ref,
                 alpha_ref, fw_ref, fb_ref, o_ref):
    q = q_ref[0]
    Rm = R_ref[0]
    kp = kp_ref[0]
    ti = ti_ref[0]
    vT = v_ref[0]
    scale = float(D_INNER) ** 0.5 + EPS
    dn11 = (((1,), (1,)), ((), ()))

    sreg = lax.dot_general(q, rk_ref[0], dn11,
                           preferred_element_type=F32) / scale
    attn = _softmax(sreg, axis=1)
    cont = jnp.dot(attn, rv_ref[0], preferred_element_type=F32)

    sall = lax.dot_general(q, kp, dn11,
                           preferred_element_type=F32) / scale
    sel = jnp.zeros((NT, M), dtype=F32)
    tsel = jnp.zeros((NT, M), dtype=F32)
    for r in range(K):
        sel = sel + Rm[:, r:r + 1] * sall[:, r * M:(r + 1) * M]
        tsel = tsel + Rm[:, r:r + 1] * ti[r:r + 1, :]

    iota_m = _fiota((NT, M), 1)
    work = sel
    topv = []
    topi = []
    for _ in range(TOPK):
        val = jnp.max(work, axis=1, keepdims=True)
        idx = jnp.min(jnp.where(work == val, iota_m, float(M)), axis=1,
                      keepdims=True)
        topv.append(val)
        topi.append(idx)
        work = jnp.where(iota_m == idx, NEG, work)
    att = _softmax(jnp.concatenate(topv, axis=1), axis=1)

    iota_n = _fiota((NT, Nd), 1)
    acc = jnp.zeros((NT, Nd), dtype=F32)
    for j in range(TOPK):
        nsrc = jnp.sum(tsel * (iota_m == topi[j]), axis=1,
                       keepdims=True)
        acc = acc + att[:, j:j + 1] * (iota_n == nsrc).astype(F32)
    sparse = jnp.dot(acc, vT, preferred_element_type=F32)

    a = jax.nn.sigmoid(alpha_ref[0, 0])
    comb = a * cont + (1.0 - a) * sparse
    fused = lax.dot_general(comb, fw_ref[0], dn11,
                            preferred_element_type=F32) + fb_ref[0]
    o_ref[0] = fused


def _interp_mat(rows, cols, row0):
    i = _fiota((rows, cols), 0) + row0
    j = _fiota((rows, cols), 1)
    src = i * (1.0 / DS) - 0.375
    i0 = jnp.floor(src)
    w1 = src - i0
    j0 = jnp.clip(i0, 0.0, float(cols - 1))
    j1 = jnp.clip(i0 + 1.0, 0.0, float(cols - 1))
    return (j == j0).astype(F32) * (1.0 - w1) + (j == j1).astype(F32) * w1


def _tr_kernel(f_ref, o_ref):
    f = f_ref[0]
    o_ref[0] = f.T.reshape(C, Hd, Wd).transpose(1, 0, 2).reshape(Hd, C * Wd)


def _up_kernel(f2_ref, x_ref, lnw_ref, lnb_ref, o_ref):
    h0 = (pl.program_id(1) * HT).astype(F32)
    Uw = _interp_mat(W, Wd, 0.0)
    Uh = _interp_mat(HT, Hd, h0)

    f2 = f2_ref[0]
    t1 = jnp.dot(Uh, f2, preferred_element_type=F32)
    t1 = t1.reshape(HT, C, Wd).transpose(1, 0, 2).reshape(C * HT, Wd)
    t2 = lax.dot_general(t1, Uw, (((1,), (1,)), ((), ())),
                         preferred_element_type=F32)
    y = t2.reshape(C, HT, W)

    mu = jnp.mean(y, axis=0, keepdims=True)
    d = y - mu
    var = jnp.mean(d * d, axis=0, keepdims=True)
    yn = d * lax.rsqrt(var + 1e-5)
    yn = yn * lnw_ref[...].reshape(C, 1, 1) + lnb_ref[...].reshape(C, 1, 1)
    o_ref[0] = x_ref[0] + yn


def kernel(x, sem_w, sem_b, q_w, k_w, v_w, fuse_w, fuse_b, alpha, ln_w, ln_b):
    f32 = jnp.float32
    x = x.astype(f32)
    sem_b2 = sem_b.reshape(1, K).astype(f32)
    fuse_b2 = fuse_b.reshape(1, C).astype(f32)
    alpha2 = alpha.reshape(1, 1).astype(f32)
    ln_w2 = ln_w.reshape(C, 1).astype(f32)
    ln_b2 = ln_b.reshape(C, 1).astype(f32)

    xf = pl.pallas_call(
        _dn_kernel,
        grid=(B, C // CT),
        in_specs=[pl.BlockSpec((1, CT, H, W), lambda b, c: (b, c, 0, 0))],
        out_specs=pl.BlockSpec((1, CT, Nd), lambda b, c: (b, c, 0)),
        out_shape=jax.ShapeDtypeStruct((B, C, Nd), f32),
    )(x)

    full = lambda b: (b, 0, 0)
    wspec = lambda s: pl.BlockSpec((1,) + s, lambda b: (0,) + (0,) * len(s))
    qT, vT, rk, rv, R, kp, ti = pl.pallas_call(
        _global_kernel,
        grid=(B,),
        in_specs=[
            pl.BlockSpec((1, C, Nd), full),
            wspec((K, C)), wspec((K, 1)), wspec((D_INNER, C)),
            wspec((D_INNER, C)), wspec((C, C)),
        ],
        out_specs=[
            pl.BlockSpec((1, Nd, D_INNER), full),
            pl.BlockSpec((1, Nd, C), full),
            pl.BlockSpec((1, K, D_INNER), full),
            pl.BlockSpec((1, K, C), full),
            pl.BlockSpec((1, Nd, K), full),
            pl.BlockSpec((1, K * M, D_INNER), full),
            pl.BlockSpec((1, K, M), full),
        ],
        out_shape=[
            jax.ShapeDtypeStruct((B, Nd, D_INNER), f32),
            jax.ShapeDtypeStruct((B, Nd, C), f32),
            jax.ShapeDtypeStruct((B, K, D_INNER), f32),
            jax.ShapeDtypeStruct((B, K, C), f32),
            jax.ShapeDtypeStruct((B, Nd, K), f32),
            jax.ShapeDtypeStruct((B, K * M, D_INNER), f32),
            jax.ShapeDtypeStruct((B, K, M), f32),
        ],
        scratch_shapes=[pltpu.VMEM((K, Nd), f32), pltpu.VMEM((K, M), f32)],
    )(xf, sem_w.reshape(1, K, C), sem_b2.reshape(1, K, 1),
      q_w.reshape(1, D_INNER, C), k_w.reshape(1, D_INNER, C),
      v_w.reshape(1, C, C))

    tile = lambda b, t: (b, t, 0)
    w2 = lambda s: pl.BlockSpec((1,) + s, lambda b, t: (0,) + (0,) * len(s))
    fullc = lambda b, t: (b, 0, 0)
    fused = pl.pallas_call(
        _attn_kernel,
        grid=(B, Nd // NT),
        in_specs=[
            pl.BlockSpec((1, NT, D_INNER), tile),
            pl.BlockSpec((1, NT, K), tile),
            pl.BlockSpec((1, K * M, D_INNER), fullc),
            pl.BlockSpec((1, K, M), fullc),
            pl.BlockSpec((1, K, D_INNER), fullc),
            pl.BlockSpec((1, K, C), fullc),
            pl.BlockSpec((1, Nd, C), fullc),
            w2((1, 1)), w2((C, C)), w2((1, C)),
        ],
        out_specs=pl.BlockSpec((1, NT, C), lambda b, t: (b, t, 0)),
        out_shape=jax.ShapeDtypeStruct((B, Nd, C), f32),
    )(qT, R, kp, ti, rk, rv, vT, alpha2.reshape(1, 1, 1),
      fuse_w.reshape(1, C, C), fuse_b2.reshape(1, 1, C))

    f2 = pl.pallas_call(
        _tr_kernel,
        grid=(B,),
        in_specs=[pl.BlockSpec((1, Nd, C), full)],
        out_specs=pl.BlockSpec((1, Hd, C * Wd), full),
        out_shape=jax.ShapeDtypeStruct((B, Hd, C * Wd), f32),
    )(fused)

    out = pl.pallas_call(
        _up_kernel,
        grid=(B, H // HT),
        in_specs=[
            pl.BlockSpec((1, Hd, C * Wd), lambda b, h: (b, 0, 0)),
            pl.BlockSpec((1, C, HT, W), lambda b, h: (b, 0, h, 0)),
            pl.BlockSpec((C, 1), lambda b, h: (0, 0)),
            pl.BlockSpec((C, 1), lambda b, h: (0, 0)),
        ],
        out_specs=pl.BlockSpec((1, C, HT, W), lambda b, h: (b, 0, h, 0)),
        out_shape=jax.ShapeDtypeStruct((B, C, H, W), f32),
    )(f2, x, ln_w2, ln_b2)
    return out


# speedup vs baseline: 5.3793x; 1.0460x over previous
import jax
import jax.numpy as jnp
from jax import lax
from jax.experimental import pallas as pl
from jax.experimental.pallas import tpu as pltpu

B, C, H, W = 2, 192, 224, 224
K = 16
TOPK = 8
DS = 4
EPS = 1e-6
D_INNER = C // 4
Hd, Wd = H // DS, W // DS
Nd = Hd * Wd
M = 256
CT = 32
NT = 448
HT = 32
NEG = -1e30
F32 = jnp.float32


def _fiota(shape, dim):
    return lax.broadcasted_iota(jnp.int32, shape, dim).astype(F32)


def _dn_kernel(x_ref, o_ref):
    xb = x_ref[0]
    ip = _fiota((Wd, W), 0)
    jp = _fiota((Wd, W), 1)
    P = (jnp.floor(jp * (1.0 / DS)) == ip).astype(F32) * (1.0 / DS)
    a = xb.reshape(CT * H, W)
    t0 = lax.dot_general(a, P, (((1,), (1,)), ((), ())),
                         preferred_element_type=F32)
    t0 = t0.reshape(CT, H, Wd).transpose(1, 0, 2).reshape(H, CT * Wd)
    t1 = jnp.dot(P, t0, preferred_element_type=F32)
    t1 = t1.reshape(Hd, CT, Wd).transpose(1, 0, 2)
    o_ref[0] = t1.reshape(CT, Nd)


def _softmax(x, axis):
    m = jnp.max(x, axis=axis, keepdims=True)
    e = jnp.exp(x - m)
    return e / jnp.sum(e, axis=axis, keepdims=True)


def _global_kernel(xf_ref, semw_ref, semb_ref, qw_ref, kw_ref, vw_ref,
                   qT_ref, vT_ref, rk_ref, rv_ref, R_ref, kp_ref, ti_ref,
                   work_ref, tis_ref):
    xf = xf_ref[0]
    dn = (((0,), (1,)), ((), ()))
    sem = jnp.dot(semw_ref[0], xf, preferred_element_type=F32)
    sem = _softmax(sem + semb_ref[0], axis=0)
    qT = lax.dot_general(xf, qw_ref[0], dn, preferred_element_type=F32)
    kT = lax.dot_general(xf, kw_ref[0], dn, preferred_element_type=F32)
    vT = lax.dot_general(xf, vw_ref[0], dn, preferred_element_type=F32)
    qT_ref[0] = qT
    vT_ref[0] = vT
    dnr = (((1,), (0,)), ((), ()))
    rk_ref[0] = lax.dot_general(sem, kT, dnr, preferred_element_type=F32)
    rv_ref[0] = lax.dot_general(sem, vT, dnr, preferred_element_type=F32)

    iota_kc = _fiota((K, Nd), 0)
    mx = jnp.max(sem, axis=0, keepdims=True)
    ridx = jnp.min(jnp.where(sem == mx, iota_kc, float(K)), axis=0,
                   keepdims=True)
    R_ref[0] = (_fiota((Nd, K), 1) == ridx.T).astype(F32)

    work_ref[...] = sem
    iota_nc = _fiota((K, Nd), 1)
    lane_m = lax.broadcasted_iota(jnp.int32, (K, M), 1)

    def body(m, _):
        w = work_ref[...]
        val = jnp.max(w, axis=1, keepdims=True)
        idx = jnp.min(jnp.where(w == val, iota_nc, float(Nd)), axis=1,
                      keepdims=True)
        tis_ref[...] = jnp.where(lane_m == m, idx, tis_ref[...])
        work_ref[...] = jnp.where(iota_nc == idx, NEG, w)
        return 0

    lax.fori_loop(0, M, body, 0)
    tis = tis_ref[...]
    ti_ref[0] = tis

    iota_row = _fiota((M, Nd), 1)
    for r in range(K):
        g = (tis[r:r + 1, :].T == iota_row).astype(F32)
        kp_ref[0, r * M:(r + 1) * M, :] = jnp.dot(
            g, kT, preferred_element_type=F32)


def _attn_kernel(q_ref, R_ref, kp_ref, ti_ref, rk_ref, rv_ref, v_ref,
                 alpha_ref, fw_ref, fb_ref, o_ref):
    q = q_ref[0]
    Rm = R_ref[0]
    kp = kp_ref[0]
    ti = ti_ref[0]
    vT = v_ref[0]
    scale = float(D_INNER) ** 0.5 + EPS
    dn11 = (((1,), (1,)), ((), ()))

    sreg = lax.dot_general(q, rk_ref[0], dn11,
                           preferred_element_type=F32) / scale
    attn = _softmax(sreg, axis=1)
    cont = jnp.dot(attn, rv_ref[0], preferred_element_type=F32)

    sall = lax.dot_general(q, kp, dn11,
                           preferred_element_type=F32) / scale
    sel = jnp.zeros((NT, M), dtype=F32)
    for r in range(K):
        sel = sel + Rm[:, r:r + 1] * sall[:, r * M:(r + 1) * M]
    tsel = jnp.dot(Rm, ti, preferred_element_type=F32)

    iota_m = lax.broadcasted_iota(jnp.int32, (NT, M), 1)
    work = sel
    topv = []
    topi = []
    for _ in range(TOPK):
        val = jnp.max(work, axis=1, keepdims=True)
        idx = jnp.min(jnp.where(work == val, iota_m, M), axis=1,
                      keepdims=True)
        topv.append(val)
        topi.append(idx)
        work = jnp.where(iota_m == idx, NEG, work)
    att = _softmax(jnp.concatenate(topv, axis=1), axis=1)

    iota_n = lax.broadcasted_iota(jnp.int32, (NT, Nd), 1)
    acc = jnp.zeros((NT, Nd), dtype=F32)
    for j in range(TOPK):
        nsrc = (jnp.sum(tsel * (iota_m == topi[j]).astype(F32), axis=1,
                        keepdims=True) + 0.5).astype(jnp.int32)
        acc = acc + jnp.where(iota_n == nsrc, att[:, j:j + 1], 0.0)
    sparse = jnp.dot(acc, vT, preferred_element_type=F32)

    a = jax.nn.sigmoid(alpha_ref[0, 0])
    comb = a * cont + (1.0 - a) * sparse
    fused = lax.dot_general(comb, fw_ref[0], dn11,
                            preferred_element_type=F32) + fb_ref[0]
    o_ref[0] = fused


def _interp_mat(rows, cols, row0):
    i = _fiota((rows, cols), 0) + row0
    j = _fiota((rows, cols), 1)
    src = i * (1.0 / DS) - 0.375
    i0 = jnp.floor(src)
    w1 = src - i0
    j0 = jnp.clip(i0, 0.0, float(cols - 1))
    j1 = jnp.clip(i0 + 1.0, 0.0, float(cols - 1))
    return (j == j0).astype(F32) * (1.0 - w1) + (j == j1).astype(F32) * w1


def _tr_kernel(f_ref, o_ref):
    f = f_ref[0]
    o_ref[0] = f.T.reshape(C, Hd, Wd).transpose(1, 0, 2).reshape(Hd, C * Wd)


def _up_kernel(f2_ref, x_ref, lnw_ref, lnb_ref, o_ref):
    h0 = (pl.program_id(1) * HT).astype(F32)
    Uw = _interp_mat(W, Wd, 0.0)
    Uh = _interp_mat(HT, Hd, h0)

    f2 = f2_ref[0]
    t1 = jnp.dot(Uh, f2, preferred_element_type=F32)
    t1 = t1.reshape(HT, C, Wd).transpose(1, 0, 2).reshape(C * HT, Wd)
    t2 = lax.dot_general(t1, Uw, (((1,), (1,)), ((), ())),
                         preferred_element_type=F32)
    y = t2.reshape(C, HT, W)

    mu = jnp.mean(y, axis=0, keepdims=True)
    d = y - mu
    var = jnp.mean(d * d, axis=0, keepdims=True)
    yn = d * lax.rsqrt(var + 1e-5)
    yn = yn * lnw_ref[...].reshape(C, 1, 1) + lnb_ref[...].reshape(C, 1, 1)
    o_ref[0] = x_ref[0] + yn


def kernel(x, sem_w, sem_b, q_w, k_w, v_w, fuse_w, fuse_b, alpha, ln_w, ln_b):
    f32 = jnp.float32
    x = x.astype(f32)
    sem_b2 = sem_b.reshape(1, K).astype(f32)
    fuse_b2 = fuse_b.reshape(1, C).astype(f32)
    alpha2 = alpha.reshape(1, 1).astype(f32)
    ln_w2 = ln_w.reshape(C, 1).astype(f32)
    ln_b2 = ln_b.reshape(C, 1).astype(f32)

    xf = pl.pallas_call(
        _dn_kernel,
        grid=(B, C // CT),
        in_specs=[pl.BlockSpec((1, CT, H, W), lambda b, c: (b, c, 0, 0))],
        out_specs=pl.BlockSpec((1, CT, Nd), lambda b, c: (b, c, 0)),
        out_shape=jax.ShapeDtypeStruct((B, C, Nd), f32),
    )(x)

    full = lambda b: (b, 0, 0)
    wspec = lambda s: pl.BlockSpec((1,) + s, lambda b: (0,) + (0,) * len(s))
    qT, vT, rk, rv, R, kp, ti = pl.pallas_call(
        _global_kernel,
        grid=(B,),
        in_specs=[
            pl.BlockSpec((1, C, Nd), full),
            wspec((K, C)), wspec((K, 1)), wspec((D_INNER, C)),
            wspec((D_INNER, C)), wspec((C, C)),
        ],
        out_specs=[
            pl.BlockSpec((1, Nd, D_INNER), full),
            pl.BlockSpec((1, Nd, C), full),
            pl.BlockSpec((1, K, D_INNER), full),
            pl.BlockSpec((1, K, C), full),
            pl.BlockSpec((1, Nd, K), full),
            pl.BlockSpec((1, K * M, D_INNER), full),
            pl.BlockSpec((1, K, M), full),
        ],
        out_shape=[
            jax.ShapeDtypeStruct((B, Nd, D_INNER), f32),
            jax.ShapeDtypeStruct((B, Nd, C), f32),
            jax.ShapeDtypeStruct((B, K, D_INNER), f32),
            jax.ShapeDtypeStruct((B, K, C), f32),
            jax.ShapeDtypeStruct((B, Nd, K), f32),
            jax.ShapeDtypeStruct((B, K * M, D_INNER), f32),
            jax.ShapeDtypeStruct((B, K, M), f32),
        ],
        scratch_shapes=[pltpu.VMEM((K, Nd), f32), pltpu.VMEM((K, M), f32)],
    )(xf, sem_w.reshape(1, K, C), sem_b2.reshape(1, K, 1),
      q_w.reshape(1, D_INNER, C), k_w.reshape(1, D_INNER, C),
      v_w.reshape(1, C, C))

    tile = lambda b, t: (b, t, 0)
    w2 = lambda s: pl.BlockSpec((1,) + s, lambda b, t: (0,) + (0,) * len(s))
    fullc = lambda b, t: (b, 0, 0)
    fused = pl.pallas_call(
        _attn_kernel,
        grid=(B, Nd // NT),
        in_specs=[
            pl.BlockSpec((1, NT, D_INNER), tile),
            pl.BlockSpec((1, NT, K), tile),
            pl.BlockSpec((1, K * M, D_INNER), fullc),
            pl.BlockSpec((1, K, M), fullc),
            pl.BlockSpec((1, K, D_INNER), fullc),
            pl.BlockSpec((1, K, C), fullc),
            pl.BlockSpec((1, Nd, C), fullc),
            w2((1, 1)), w2((C, C)), w2((1, C)),
        ],
        out_specs=pl.BlockSpec((1, NT, C), lambda b, t: (b, t, 0)),
        out_shape=jax.ShapeDtypeStruct((B, Nd, C), f32),
    )(qT, R, kp, ti, rk, rv, vT, alpha2.reshape(1, 1, 1),
      fuse_w.reshape(1, C, C), fuse_b2.reshape(1, 1, C))

    f2 = pl.pallas_call(
        _tr_kernel,
        grid=(B,),
        in_specs=[pl.BlockSpec((1, Nd, C), full)],
        out_specs=pl.BlockSpec((1, Hd, C * Wd), full),
        out_shape=jax.ShapeDtypeStruct((B, Hd, C * Wd), f32),
    )(fused)

    out = pl.pallas_call(
        _up_kernel,
        grid=(B, H // HT),
        in_specs=[
            pl.BlockSpec((1, Hd, C * Wd), lambda b, h: (b, 0, 0)),
            pl.BlockSpec((1, C, HT, W), lambda b, h: (b, 0, h, 0)),
            pl.BlockSpec((C, 1), lambda b, h: (0, 0)),
            pl.BlockSpec((C, 1), lambda b, h: (0, 0)),
        ],
        out_specs=pl.BlockSpec((1, C, HT, W), lambda b, h: (b, 0, h, 0)),
        out_shape=jax.ShapeDtypeStruct((B, C, H, W), f32),
    )(f2, x, ln_w2, ln_b2)
    return out
